# Initial kernel scaffold; baseline (speedup 1.0000x reference)
import jax
import jax.numpy as jnp
from jax.experimental import pallas as pl


def kernel(tokens, num_steps, prev_steps, table_obs, table_act):
    raise NotImplementedError("write your pallas kernel here")


# speedup vs baseline: 6.8057x; 6.8057x over previous
import functools

import numpy as np
import jax
import jax.numpy as jnp
from jax import lax
from jax.experimental import pallas as pl
from jax.experimental.pallas import tpu as pltpu
from jax.experimental.pallas import tpu_sc as plsc

_B = 128
_BLOCK_SIZE = 17
_L = 128 * _BLOCK_SIZE
_D = 256
_V = 512

_NC, _NS = 2, 16
_NW = _NC * _NS
_N = _B * _L
_PER_W = _N // _NW
_CHUNK = 128
_NCHUNK = _PER_W // _CHUNK
_LANES = 16

_OFF_TILE = np.tile(
    np.where((np.arange(_L) % _BLOCK_SIZE) == (_BLOCK_SIZE - 1), _V, 0),
    _PER_W // _L,
).astype(np.int32).reshape(_NCHUNK, _CHUNK)


@functools.lru_cache(maxsize=None)
def _build_sc_embed():
    mesh = plsc.VectorSubcoreMesh(core_axis_name="c", subcore_axis_name="s")

    @functools.partial(
        pl.kernel,
        mesh=mesh,
        out_type=jax.ShapeDtypeStruct((_N, _D), jnp.float32),
        scratch_types=[
            pltpu.VMEM((_NCHUNK, _CHUNK), jnp.int32),
            pltpu.VMEM((_NCHUNK, _CHUNK), jnp.int32),
            pltpu.VMEM((_CHUNK, _D), jnp.float32),
            pltpu.VMEM((_CHUNK, _D), jnp.float32),
            pltpu.SemaphoreType.DMA,
            pltpu.SemaphoreType.DMA,
        ],
    )
    def sc_embed(table_hbm, tok_hbm, off_hbm, out_hbm,
                 idx_v, off_v, buf_a, buf_b, sem_a, sem_b):
        wid = lax.axis_index("s") * _NC + lax.axis_index("c")
        base_row = wid * _PER_W

        pltpu.sync_copy(tok_hbm.at[wid], idx_v)
        pltpu.sync_copy(off_hbm, off_v)

        def add_body(c, carry):
            for u in range(_CHUNK // _LANES):
                sl = pl.ds(u * _LANES, _LANES)
                idx_v[c, sl] = idx_v[c, sl] + off_v[c, sl]
            return carry

        lax.fori_loop(0, _NCHUNK, add_body, 0)

        def fire(c, buf, sem):
            pltpu.make_async_copy(table_hbm.at[idx_v.at[c]], buf, sem).start()

        def wait(c, buf, sem):
            pltpu.make_async_copy(table_hbm.at[idx_v.at[c]], buf, sem).wait()

        def write(c, buf):
            pltpu.sync_copy(buf, out_hbm.at[pl.ds(base_row + c * _CHUNK, _CHUNK)])

        fire(0, buf_a, sem_a)

        def loop_body(i, carry):
            c0 = 2 * i
            wait(c0, buf_a, sem_a)
            fire(c0 + 1, buf_b, sem_b)
            write(c0, buf_a)

            @pl.when(c0 + 2 < _NCHUNK)
            def _():
                fire(c0 + 2, buf_a, sem_a)

            wait(c0 + 1, buf_b, sem_b)
            write(c0 + 1, buf_b)
            return carry

        lax.fori_loop(0, _NCHUNK // 2, loop_body, 0)

    return sc_embed


def kernel(tokens, num_steps, prev_steps, table_obs, table_act):
    del num_steps, prev_steps
    table = jnp.concatenate([table_obs, table_act], axis=0)
    tok = tokens.astype(jnp.int32).reshape(_NW, _NCHUNK, _CHUNK)
    off = jnp.asarray(_OFF_TILE)
    out = _build_sc_embed()(table, tok, off)
    return out.reshape(_B, _L, _D)
